# Initial kernel scaffold; baseline (speedup 1.0000x reference)
import jax
import jax.numpy as jnp
from jax.experimental import pallas as pl


def kernel(query_up, key_up, head_idx):
    raise NotImplementedError("write your pallas kernel here")


# speedup vs baseline: 16.6364x; 16.6364x over previous
import functools

import jax
import jax.numpy as jnp
import numpy as np
from jax import lax
from jax.experimental import pallas as pl
from jax.experimental.pallas import tpu as pltpu
from jax.experimental.pallas import tpu_sc as plsc

B, L, D = 2, 2048, 64
NG, GS = 8, 8
K_MAX = 64
HW = L // 16
QW = 384
OW = 512
BQ = 256
NW = 32
RPW = (B * L) // NW

_W = np.zeros((D, NG), np.float32)
for _g in range(NG):
    for _t in range(GS):
        _W[_g * GS + _t, _g] = float(1 << _t)
_WT = np.ascontiguousarray(_W.T)

_Q = np.zeros((L, QW), np.float32)
for _j in range(L):
    _Q[_j, _j // 16] = float(1 << (_j % 16))
    _Q[_j, 128 + (_j // 16) + 1:256] = 1.0
    _Q[_j, 256] = 1.0


def _match_body(q_ref, kT_ref, w_ref, wT_ref, p_ref, out_ref):
    q = q_ref[0]
    kT = kT_ref[0]
    qb = (q > 0.0).astype(jnp.float32)
    kbT = (kT > 0.0).astype(jnp.float32)
    qc = jnp.dot(qb, w_ref[...], preferred_element_type=jnp.float32)
    kcT = jnp.dot(wT_ref[...], kbT, preferred_element_type=jnp.float32)
    m = qc[:, 0:1] == kcT[0:1, :]
    for g in range(1, NG):
        m = m | (qc[:, g:g + 1] == kcT[g:g + 1, :])
    packed = jnp.dot(m.astype(jnp.bfloat16), p_ref[...].astype(jnp.bfloat16),
                     preferred_element_type=jnp.float32)
    r_f = packed[:, 128:256]
    c64 = lax.broadcasted_iota(jnp.int32, (1, K_MAX), 1).astype(jnp.float32)
    acc = jnp.zeros((BQ, K_MAX), jnp.float32)
    for w in range(HW):
        acc = acc + (r_f[:, w:w + 1] <= c64).astype(jnp.float32)
    wc = jnp.concatenate([acc, jnp.zeros((BQ, OW - QW - K_MAX), jnp.float32)],
                         axis=1)
    out_ref[0, :, 0:QW] = packed.astype(jnp.int32)
    out_ref[0, :, QW:OW] = wc.astype(jnp.int32)


def _match_records(query_up, keyT):
    return pl.pallas_call(
        _match_body,
        grid=(B, L // BQ),
        in_specs=[
            pl.BlockSpec((1, BQ, D), lambda b, i: (b, i, 0)),
            pl.BlockSpec((1, D, L), lambda b, i: (b, 0, 0)),
            pl.BlockSpec((D, NG), lambda b, i: (0, 0)),
            pl.BlockSpec((NG, D), lambda b, i: (0, 0)),
            pl.BlockSpec((L, QW), lambda b, i: (0, 0)),
        ],
        out_specs=pl.BlockSpec((1, BQ, OW), lambda b, i: (b, i, 0)),
        out_shape=jax.ShapeDtypeStruct((B, L, OW), jnp.int32),
        compiler_params=pltpu.CompilerParams(
            dimension_semantics=("parallel", "parallel")),
    )(query_up, keyT, jnp.asarray(_W), jnp.asarray(_WT), jnp.asarray(_Q))


@functools.cache
def _build_select():
    return functools.partial(
        pl.kernel,
        mesh=plsc.VectorSubcoreMesh(core_axis_name="c", subcore_axis_name="s"),
        out_type=jax.ShapeDtypeStruct((B * L, K_MAX), jnp.int32),
        scratch_types=[
            pltpu.VMEM((RPW, OW), jnp.int32),
            pltpu.VMEM((RPW, K_MAX), jnp.int32),
        ],
    )(_select_body)


def _gdn():
    return lax.GatherDimensionNumbers(
        offset_dims=(), collapsed_slice_dims=(0,), start_index_map=(0,))


def _vgather(vec, idx):
    return lax.gather(vec, idx[:, None], _gdn(), slice_sizes=(1,),
                      mode=lax.GatherScatterMode.PROMISE_IN_BOUNDS)


def _select_body(rec_hbm, out_hbm, rec_v, out_v):
    wid = lax.axis_index("s") * 2 + lax.axis_index("c")
    base = wid * RPW
    pltpu.sync_copy(rec_hbm.at[pl.ds(base, RPW)], rec_v)
    iota = lax.iota(jnp.int32, 16)

    def row_body(r, carry):
        hvs = [rec_v[r, pl.ds(16 * b, 16)] for b in range(8)]
        rvs = [rec_v[r, pl.ds(128 + 16 * b, 16)] for b in range(8)]
        total = rec_v[r, pl.ds(256, 16)][0]
        for k in range(K_MAX // 16):
            c = iota + (16 * k)
            wv = rec_v[r, pl.ds(QW + 16 * k, 16)] - 1
            low = wv & 15
            hi = wv >> 4
            rv = jnp.zeros((16,), jnp.int32)
            vv = jnp.zeros((16,), jnp.int32)
            for b in range(8):
                pred = hi == b
                rv = jnp.where(pred, _vgather(rvs[b], low), rv)
                vv = jnp.where(pred, _vgather(hvs[b], low), vv)
            c_loc = c - rv
            cnt = jnp.zeros((16,), jnp.int32)
            pos = jnp.zeros((16,), jnp.int32)
            for t in range(16):
                bt = (vv >> t) & 1
                hit = (cnt == c_loc) & (bt == 1)
                pos = jnp.where(hit, t, pos)
                cnt = cnt + bt
            res = jnp.where(c < total, wv * 16 + pos, -1)
            out_v[r, pl.ds(16 * k, 16)] = res
        return carry

    lax.fori_loop(0, RPW, row_body, jnp.int32(0))
    pltpu.sync_copy(out_v, out_hbm.at[pl.ds(base, RPW)])


def kernel(query_up, key_up, head_idx):
    del head_idx
    keyT = jnp.swapaxes(key_up, 1, 2)
    rec = _match_records(query_up, keyT)
    cand = _build_select()(rec.reshape(B * L, OW))
    return cand.reshape(B, L, K_MAX)


# speedup vs baseline: 18.8950x; 1.1358x over previous
import functools

import jax
import jax.numpy as jnp
import numpy as np
from jax import lax
from jax.experimental import pallas as pl
from jax.experimental.pallas import tpu as pltpu
from jax.experimental.pallas import tpu_sc as plsc

B, L, D = 2, 2048, 64
NG, GS = 8, 8
K_MAX = 64
HW = L // 16
QW = 384
OW = 256
BQ = 256
NW = 32
RPW = (B * L) // NW

_W = np.zeros((D, NG), np.float32)
for _g in range(NG):
    for _t in range(GS):
        _W[_g * GS + _t, _g] = float(1 << _t)
_WT = np.ascontiguousarray(_W.T)

_Q = np.zeros((L, QW), np.float32)
for _j in range(L):
    _Q[_j, _j // 16] = float(1 << (_j % 16))
    _Q[_j, 128 + (_j // 16) + 1:256] = 1.0
    _Q[_j, 256] = 1.0


def _match_body(q_ref, k_ref, w_ref, wT_ref, p_ref, out_ref):
    q = q_ref[0]
    k = k_ref[0]
    qb = (q > 0.0).astype(jnp.float32)
    kb = (k > 0.0).astype(jnp.float32)
    qc = jnp.dot(qb, w_ref[...],
                 preferred_element_type=jnp.float32).astype(jnp.bfloat16)
    kcT = lax.dot_general(wT_ref[...], kb, (((1,), (1,)), ((), ())),
                          preferred_element_type=jnp.float32
                          ).astype(jnp.bfloat16)
    m = qc[:, 0:1] == kcT[0:1, :]
    for g in range(1, NG):
        m = m | (qc[:, g:g + 1] == kcT[g:g + 1, :])
    packed = jnp.dot(m.astype(jnp.bfloat16), p_ref[...].astype(jnp.bfloat16),
                     preferred_element_type=jnp.float32)
    hw_i = packed[:, 0:HW].astype(jnp.int32)
    r_f = packed[:, HW:2 * HW]
    t_i = jnp.minimum(packed[:, 2 * HW:2 * HW + 1], 64.0).astype(jnp.int32)
    rec = hw_i | (jnp.minimum(r_f, 64.0).astype(jnp.int32) << 16)
    rcb = jnp.minimum(r_f, 65.0).astype(jnp.bfloat16)
    c64 = lax.broadcasted_iota(jnp.int32, (1, K_MAX), 1).astype(jnp.bfloat16)
    acc = jnp.zeros((BQ, K_MAX), jnp.bfloat16)
    for w in range(HW):
        acc = acc + (rcb[:, w:w + 1] <= c64).astype(jnp.bfloat16)
    half2 = jnp.concatenate(
        [acc.astype(jnp.int32), t_i,
         jnp.zeros((BQ, OW - HW - K_MAX - 1), jnp.int32)], axis=1)
    out_ref[0, :, 0:HW] = rec
    out_ref[0, :, HW:OW] = half2


def _match_records(query_up, key_up):
    return pl.pallas_call(
        _match_body,
        grid=(B, L // BQ),
        in_specs=[
            pl.BlockSpec((1, BQ, D), lambda b, i: (b, i, 0)),
            pl.BlockSpec((1, L, D), lambda b, i: (b, 0, 0)),
            pl.BlockSpec((D, NG), lambda b, i: (0, 0)),
            pl.BlockSpec((NG, D), lambda b, i: (0, 0)),
            pl.BlockSpec((L, QW), lambda b, i: (0, 0)),
        ],
        out_specs=pl.BlockSpec((1, BQ, OW), lambda b, i: (b, i, 0)),
        out_shape=jax.ShapeDtypeStruct((B, L, OW), jnp.int32),
        compiler_params=pltpu.CompilerParams(
            dimension_semantics=("parallel", "parallel")),
    )(query_up, key_up, jnp.asarray(_W), jnp.asarray(_WT), jnp.asarray(_Q))


@functools.cache
def _build_select():
    return functools.partial(
        pl.kernel,
        mesh=plsc.VectorSubcoreMesh(core_axis_name="c", subcore_axis_name="s"),
        out_type=jax.ShapeDtypeStruct((B * L, K_MAX), jnp.int32),
        scratch_types=[
            pltpu.VMEM((RPW, OW), jnp.int32),
            pltpu.VMEM((RPW, K_MAX), jnp.int32),
        ],
    )(_select_body)


def _gdn():
    return lax.GatherDimensionNumbers(
        offset_dims=(), collapsed_slice_dims=(0,), start_index_map=(0,))


def _vgather(vec, idx):
    return lax.gather(vec, idx[:, None], _gdn(), slice_sizes=(1,),
                      mode=lax.GatherScatterMode.PROMISE_IN_BOUNDS)


def _select_body(rec_hbm, out_hbm, rec_v, out_v):
    wid = lax.axis_index("s") * 2 + lax.axis_index("c")
    base = wid * RPW
    pltpu.sync_copy(rec_hbm.at[pl.ds(base, RPW)], rec_v)
    iota = lax.iota(jnp.int32, 16)

    def row_body(r, carry):
        cvs = [rec_v[r, pl.ds(16 * b, 16)] for b in range(8)]
        total = rec_v[r, pl.ds(HW + K_MAX, 16)][0]
        for kk in range(K_MAX // 16):
            c = iota + (16 * kk)
            wv = rec_v[r, pl.ds(HW + 16 * kk, 16)] - 1
            low = wv & 15
            hi = wv >> 4
            g = jnp.zeros((16,), jnp.int32)
            for b in range(8):
                g = jnp.where(hi == b, _vgather(cvs[b], low), g)
            vv = g & 0xFFFF
            c_loc = c - (g >> 16)
            cnt = jnp.zeros((16,), jnp.int32)
            pos = jnp.zeros((16,), jnp.int32)
            for t in range(16):
                bt = (vv >> t) & 1
                hit = (cnt == c_loc) & (bt == 1)
                pos = jnp.where(hit, t, pos)
                cnt = cnt + bt
            res = jnp.where(c < total, wv * 16 + pos, -1)
            out_v[r, pl.ds(16 * kk, 16)] = res
        return carry

    lax.fori_loop(0, RPW, row_body, jnp.int32(0))
    pltpu.sync_copy(out_v, out_hbm.at[pl.ds(base, RPW)])


def kernel(query_up, key_up, head_idx):
    del head_idx
    rec = _match_records(query_up, key_up)
    cand = _build_select()(rec.reshape(B * L, OW))
    return cand.reshape(B, L, K_MAX)


# speedup vs baseline: 23.8441x; 1.2619x over previous
import functools

import jax
import jax.numpy as jnp
import numpy as np
from jax import lax
from jax.experimental import pallas as pl
from jax.experimental.pallas import tpu as pltpu
from jax.experimental.pallas import tpu_sc as plsc

B, L, D = 2, 2048, 64
NG, GS = 8, 8
K_MAX = 64
HW = L // 16
QW = 384
OW = 256
BQ = 256
NW = 32
RPW = (B * L) // NW

_W = np.zeros((D, NG), np.float32)
for _g in range(NG):
    for _t in range(GS):
        _W[_g * GS + _t, _g] = float(1 << _t)
_WT = np.ascontiguousarray(_W.T)

_Q = np.zeros((L, QW), np.float32)
for _j in range(L):
    _Q[_j, _j // 16] = float(1 << (_j % 16))
    _Q[_j, 128 + (_j // 16) + 1:256] = 1.0
    _Q[_j, 256] = 1.0
    _Q[_j, 257 + (_j // 32) + 1:321] = 1.0


def _match_body(q_ref, k_ref, w_ref, wT_ref, p_ref, out_ref):
    q = q_ref[0]
    k = k_ref[0]
    qb = (q > 0.0).astype(jnp.float32)
    kb = (k > 0.0).astype(jnp.float32)
    qc = jnp.dot(qb, w_ref[...],
                 preferred_element_type=jnp.float32).astype(jnp.bfloat16)
    kcT = lax.dot_general(wT_ref[...], kb, (((1,), (1,)), ((), ())),
                          preferred_element_type=jnp.float32
                          ).astype(jnp.bfloat16)
    m = qc[:, 0:1] == kcT[0:1, :]
    for g in range(1, NG):
        m = m | (qc[:, g:g + 1] == kcT[g:g + 1, :])
    packed = jnp.dot(m.astype(jnp.bfloat16), p_ref[...].astype(jnp.bfloat16),
                     preferred_element_type=jnp.float32)
    hw_i = packed[:, 0:HW].astype(jnp.int32)
    r_f = packed[:, HW:2 * HW]
    t_i = jnp.minimum(packed[:, 2 * HW:2 * HW + 1], 64.0).astype(jnp.int32)
    rec = hw_i | (jnp.minimum(r_f, 64.0).astype(jnp.int32) << 16)
    rc32 = jnp.minimum(packed[:, 2 * HW + 1:2 * HW + 1 + K_MAX], 65.0)
    c64 = lax.broadcasted_iota(jnp.int32, (1, K_MAX), 1).astype(jnp.float32)
    acc = jnp.zeros((BQ, K_MAX), jnp.float32)
    for w in range(K_MAX):
        acc = acc + (rc32[:, w:w + 1] <= c64).astype(jnp.float32)
    half2 = jnp.concatenate(
        [acc.astype(jnp.int32), t_i,
         jnp.zeros((BQ, OW - HW - K_MAX - 1), jnp.int32)], axis=1)
    out_ref[0, :, 0:HW] = rec
    out_ref[0, :, HW:OW] = half2


def _match_records(query_up, key_up):
    return pl.pallas_call(
        _match_body,
        grid=(B, L // BQ),
        in_specs=[
            pl.BlockSpec((1, BQ, D), lambda b, i: (b, i, 0)),
            pl.BlockSpec((1, L, D), lambda b, i: (b, 0, 0)),
            pl.BlockSpec((D, NG), lambda b, i: (0, 0)),
            pl.BlockSpec((NG, D), lambda b, i: (0, 0)),
            pl.BlockSpec((L, QW), lambda b, i: (0, 0)),
        ],
        out_specs=pl.BlockSpec((1, BQ, OW), lambda b, i: (b, i, 0)),
        out_shape=jax.ShapeDtypeStruct((B, L, OW), jnp.int32),
        compiler_params=pltpu.CompilerParams(
            dimension_semantics=("parallel", "parallel")),
    )(query_up, key_up, jnp.asarray(_W), jnp.asarray(_WT), jnp.asarray(_Q))


@functools.cache
def _build_select():
    return functools.partial(
        pl.kernel,
        mesh=plsc.VectorSubcoreMesh(core_axis_name="c", subcore_axis_name="s"),
        out_type=jax.ShapeDtypeStruct((B * L, K_MAX), jnp.int32),
        scratch_types=[
            pltpu.VMEM((RPW, OW), jnp.int32),
            pltpu.VMEM((RPW, K_MAX), jnp.int32),
        ],
    )(_select_body)


def _gdn():
    return lax.GatherDimensionNumbers(
        offset_dims=(), collapsed_slice_dims=(0,), start_index_map=(0,))


def _vgather(vec, idx):
    return lax.gather(vec, idx[:, None], _gdn(), slice_sizes=(1,),
                      mode=lax.GatherScatterMode.PROMISE_IN_BOUNDS)


def _select_body(rec_hbm, out_hbm, rec_v, out_v):
    wid = lax.axis_index("s") * 2 + lax.axis_index("c")
    base = wid * RPW
    pltpu.sync_copy(rec_hbm.at[pl.ds(base, RPW)], rec_v)
    iota = lax.iota(jnp.int32, 16)

    def row_body(r, carry):
        cvs = [rec_v[r, pl.ds(16 * b, 16)] for b in range(8)]
        total = rec_v[r, pl.ds(HW + K_MAX, 16)][0]
        for kk in range(K_MAX // 16):
            c = iota + (16 * kk)
            wv = rec_v[r, pl.ds(HW + 16 * kk, 16)] - 1
            low = (wv << 1) & 15
            hi = wv >> 3
            g0 = jnp.zeros((16,), jnp.int32)
            g1 = jnp.zeros((16,), jnp.int32)
            for b in range(8):
                pred = hi == b
                g0 = jnp.where(pred, _vgather(cvs[b], low), g0)
                g1 = jnp.where(pred, _vgather(cvs[b], low + 1), g1)
            use1 = (g1 >> 16) <= c
            g = jnp.where(use1, g1, g0)
            whw = (wv << 1) + jnp.where(use1, 1, 0)
            vv = g & 0xFFFF
            c_loc = c - (g >> 16)
            cnt = jnp.zeros((16,), jnp.int32)
            pos = jnp.zeros((16,), jnp.int32)
            for t in range(16):
                bt = (vv >> t) & 1
                hit = (cnt == c_loc) & (bt == 1)
                pos = jnp.where(hit, t, pos)
                cnt = cnt + bt
            res = jnp.where(c < total, whw * 16 + pos, -1)
            out_v[r, pl.ds(16 * kk, 16)] = res
        return carry

    lax.fori_loop(0, RPW, row_body, jnp.int32(0))
    pltpu.sync_copy(out_v, out_hbm.at[pl.ds(base, RPW)])


def kernel(query_up, key_up, head_idx):
    del head_idx
    rec = _match_records(query_up, key_up)
    cand = _build_select()(rec.reshape(B * L, OW))
    return cand.reshape(B, L, K_MAX)


# speedup vs baseline: 25.0964x; 1.0525x over previous
import functools

import jax
import jax.numpy as jnp
import numpy as np
from jax import lax
from jax.experimental import pallas as pl
from jax.experimental.pallas import tpu as pltpu
from jax.experimental.pallas import tpu_sc as plsc

B, L, D = 2, 2048, 64
NG, GS = 8, 8
K_MAX = 64
HW = L // 16
QW = 384
OW = 256
BQ = 256
NW = 32
RPW = L // NW

_W = np.zeros((D, NG), np.float32)
for _g in range(NG):
    for _t in range(GS):
        _W[_g * GS + _t, _g] = float(1 << _t)
_WT = np.ascontiguousarray(_W.T)

_Q = np.zeros((L, QW), np.float32)
for _j in range(L):
    _Q[_j, _j // 16] = float(1 << (_j % 16))
    _Q[_j, 128 + (_j // 16) + 1:256] = 1.0
    _Q[_j, 256] = 1.0
    _Q[_j, 257 + (_j // 32) + 1:321] = 1.0


def _match_body(q_ref, k_ref, w_ref, wT_ref, p_ref, out_ref):
    q = q_ref[...]
    k = k_ref[...]
    qb = (q > 0.0).astype(jnp.float32)
    kb = (k > 0.0).astype(jnp.float32)
    qc = jnp.dot(qb, w_ref[...],
                 preferred_element_type=jnp.float32).astype(jnp.bfloat16)
    kcT = lax.dot_general(wT_ref[...], kb, (((1,), (1,)), ((), ())),
                          preferred_element_type=jnp.float32
                          ).astype(jnp.bfloat16)
    m = qc[:, 0:1] == kcT[0:1, :]
    for g in range(1, NG):
        m = m | (qc[:, g:g + 1] == kcT[g:g + 1, :])
    packed = jnp.dot(m.astype(jnp.bfloat16), p_ref[...].astype(jnp.bfloat16),
                     preferred_element_type=jnp.float32)
    hw_i = packed[:, 0:HW].astype(jnp.int32)
    r_f = packed[:, HW:2 * HW]
    t_i = jnp.minimum(packed[:, 2 * HW:2 * HW + 1], 64.0).astype(jnp.int32)
    rec = hw_i | (jnp.minimum(r_f, 64.0).astype(jnp.int32) << 16)
    rc32 = jnp.minimum(packed[:, 2 * HW + 1:2 * HW + 1 + K_MAX], 65.0)
    c64 = lax.broadcasted_iota(jnp.int32, (1, K_MAX), 1).astype(jnp.float32)
    acc = jnp.zeros((BQ, K_MAX), jnp.float32)
    for w in range(K_MAX):
        acc = acc + (rc32[:, w:w + 1] <= c64).astype(jnp.float32)
    half2 = jnp.concatenate(
        [acc.astype(jnp.int32), t_i,
         jnp.zeros((BQ, OW - HW - K_MAX - 1), jnp.int32)], axis=1)
    out_ref[:, 0:HW] = rec
    out_ref[:, HW:OW] = half2


def _match_records(query_1b, key_1b):
    return pl.pallas_call(
        _match_body,
        grid=(L // BQ,),
        in_specs=[
            pl.BlockSpec((BQ, D), lambda i: (i, 0)),
            pl.BlockSpec((L, D), lambda i: (0, 0)),
            pl.BlockSpec((D, NG), lambda i: (0, 0)),
            pl.BlockSpec((NG, D), lambda i: (0, 0)),
            pl.BlockSpec((L, QW), lambda i: (0, 0)),
        ],
        out_specs=pl.BlockSpec((BQ, OW), lambda i: (i, 0)),
        out_shape=jax.ShapeDtypeStruct((L, OW), jnp.int32),
        compiler_params=pltpu.CompilerParams(
            dimension_semantics=("parallel",)),
    )(query_1b, key_1b, jnp.asarray(_W), jnp.asarray(_WT), jnp.asarray(_Q))


@functools.cache
def _build_select():
    return functools.partial(
        pl.kernel,
        mesh=plsc.VectorSubcoreMesh(core_axis_name="c", subcore_axis_name="s"),
        out_type=jax.ShapeDtypeStruct((L, K_MAX), jnp.int32),
        scratch_types=[
            pltpu.VMEM((RPW, OW), jnp.int32),
            pltpu.VMEM((RPW, K_MAX), jnp.int32),
        ],
    )(_select_body)


def _gdn():
    return lax.GatherDimensionNumbers(
        offset_dims=(), collapsed_slice_dims=(0,), start_index_map=(0,))


def _vgather(vec, idx):
    return lax.gather(vec, idx[:, None], _gdn(), slice_sizes=(1,),
                      mode=lax.GatherScatterMode.PROMISE_IN_BOUNDS)


def _select_body(rec_hbm, out_hbm, rec_v, out_v):
    wid = lax.axis_index("s") * 2 + lax.axis_index("c")
    base = wid * RPW
    pltpu.sync_copy(rec_hbm.at[pl.ds(base, RPW)], rec_v)
    iota = lax.iota(jnp.int32, 16)

    def row_body(r, carry):
        cvs = [rec_v[r, pl.ds(16 * b, 16)] for b in range(8)]
        total = rec_v[r, pl.ds(HW + K_MAX, 16)][0]
        for kk in range(K_MAX // 16):
            c = iota + (16 * kk)
            wv = rec_v[r, pl.ds(HW + 16 * kk, 16)] - 1
            low = (wv << 1) & 15
            hi = wv >> 3
            g0 = jnp.zeros((16,), jnp.int32)
            g1 = jnp.zeros((16,), jnp.int32)
            for b in range(8):
                pred = hi == b
                g0 = jnp.where(pred, _vgather(cvs[b], low), g0)
                g1 = jnp.where(pred, _vgather(cvs[b], low + 1), g1)
            use1 = (g1 >> 16) <= c
            g = jnp.where(use1, g1, g0)
            whw = (wv << 1) + jnp.where(use1, 1, 0)
            vv = g & 0xFFFF
            c_loc = c - (g >> 16)
            cnt = jnp.zeros((16,), jnp.int32)
            pos = jnp.zeros((16,), jnp.int32)
            for t in range(16):
                bt = (vv >> t) & 1
                hit = (cnt == c_loc) & (bt == 1)
                pos = jnp.where(hit, t, pos)
                cnt = cnt + bt
            res = jnp.where(c < total, whw * 16 + pos, -1)
            out_v[r, pl.ds(16 * kk, 16)] = res
        return carry

    lax.fori_loop(0, RPW, row_body, jnp.int32(0))
    pltpu.sync_copy(out_v, out_hbm.at[pl.ds(base, RPW)])


def kernel(query_up, key_up, head_idx):
    del head_idx
    sel = _build_select()
    outs = []
    for b in range(B):
        rec = _match_records(query_up[b], key_up[b])
        outs.append(sel(rec))
    return jnp.stack(outs)


# speedup vs baseline: 31.1853x; 1.2426x over previous
import functools

import jax
import jax.numpy as jnp
import numpy as np
from jax import lax
from jax.experimental import pallas as pl
from jax.experimental.pallas import tpu as pltpu
from jax.experimental.pallas import tpu_sc as plsc

B, L, D = 2, 2048, 64
NG, GS = 8, 8
K_MAX = 64
HW = L // 16
QW = 256
OW = 256
BQ = 256
NW = 32
RPW = L // NW

_W = np.zeros((D, NG), np.float32)
for _g in range(NG):
    for _t in range(GS):
        _W[_g * GS + _t, _g] = float(1 << _t)
_WT = np.ascontiguousarray(_W.T)

_Q = np.zeros((L, QW), np.float32)
for _j in range(L):
    _w = _j // 16
    if _w % 2 == 0:
        _Q[_j, _j // 32] = float(1 << (_j % 16))
    else:
        _Q[_j, 64 + _j // 32] = float(1 << (_j % 16))
    _Q[_j, 128 + (_j // 64) + 1:160] = 1.0
    _u, _e = _w // 4, _w % 4
    if _e < 3:
        _Q[_j, 160 + _u] = float(1 << (5 * _e))
    _Q[_j, 192] = 1.0


def _match_body(q_ref, k_ref, w_ref, wT_ref, p_ref, out_ref):
    q = q_ref[...]
    k = k_ref[...]
    qb = (q > 0.0).astype(jnp.float32)
    kb = (k > 0.0).astype(jnp.float32)
    qc = jnp.dot(qb, w_ref[...],
                 preferred_element_type=jnp.float32).astype(jnp.bfloat16)
    kcT = lax.dot_general(wT_ref[...], kb, (((1,), (1,)), ((), ())),
                          preferred_element_type=jnp.float32
                          ).astype(jnp.bfloat16)
    m = qc[:, 0:1] == kcT[0:1, :]
    for g in range(1, NG):
        m = m | (qc[:, g:g + 1] == kcT[g:g + 1, :])
    packed = jnp.dot(m.astype(jnp.bfloat16), p_ref[...].astype(jnp.bfloat16),
                     preferred_element_type=jnp.float32)
    hwpair = (packed[:, 0:64].astype(jnp.int32)
              | (packed[:, 64:128].astype(jnp.int32) << 16))
    r64f = packed[:, 128:160]
    rq = (jnp.minimum(r64f, 64.0).astype(jnp.int32)
          | (packed[:, 160:192].astype(jnp.int32) << 7))
    t_i = jnp.minimum(packed[:, 192:193], 64.0).astype(jnp.int32)
    rc = jnp.minimum(r64f, 65.0)
    c64 = lax.broadcasted_iota(jnp.int32, (1, K_MAX), 1).astype(jnp.float32)
    acc = jnp.zeros((BQ, K_MAX), jnp.float32)
    for w in range(32):
        acc = acc + (rc[:, w:w + 1] <= c64).astype(jnp.float32)
    half1 = jnp.concatenate(
        [hwpair, rq, jnp.zeros((BQ, 32), jnp.int32)], axis=1)
    half2 = jnp.concatenate(
        [acc.astype(jnp.int32), t_i, jnp.zeros((BQ, 63), jnp.int32)], axis=1)
    out_ref[:, 0:HW] = half1
    out_ref[:, HW:OW] = half2


def _match_records(query_1b, key_1b):
    return pl.pallas_call(
        _match_body,
        grid=(L // BQ,),
        in_specs=[
            pl.BlockSpec((BQ, D), lambda i: (i, 0)),
            pl.BlockSpec((L, D), lambda i: (0, 0)),
            pl.BlockSpec((D, NG), lambda i: (0, 0)),
            pl.BlockSpec((NG, D), lambda i: (0, 0)),
            pl.BlockSpec((L, QW), lambda i: (0, 0)),
        ],
        out_specs=pl.BlockSpec((BQ, OW), lambda i: (i, 0)),
        out_shape=jax.ShapeDtypeStruct((L, OW), jnp.int32),
        compiler_params=pltpu.CompilerParams(
            dimension_semantics=("parallel",)),
    )(query_1b, key_1b, jnp.asarray(_W), jnp.asarray(_WT), jnp.asarray(_Q))


@functools.cache
def _build_select():
    return functools.partial(
        pl.kernel,
        mesh=plsc.VectorSubcoreMesh(core_axis_name="c", subcore_axis_name="s"),
        out_type=jax.ShapeDtypeStruct((L, K_MAX), jnp.int32),
        scratch_types=[
            pltpu.VMEM((RPW, OW), jnp.int32),
            pltpu.VMEM((RPW, K_MAX), jnp.int32),
        ],
    )(_select_body)


def _gdn():
    return lax.GatherDimensionNumbers(
        offset_dims=(), collapsed_slice_dims=(0,), start_index_map=(0,))


def _vgather(vec, idx):
    return lax.gather(vec, idx[:, None], _gdn(), slice_sizes=(1,),
                      mode=lax.GatherScatterMode.PROMISE_IN_BOUNDS)


def _select_body(rec_hbm, out_hbm, rec_v, out_v):
    wid = lax.axis_index("s") * 2 + lax.axis_index("c")
    base = wid * RPW
    pltpu.sync_copy(rec_hbm.at[pl.ds(base, RPW)], rec_v)
    iota = lax.iota(jnp.int32, 16)
    pop4 = ((iota & 1) + ((iota >> 1) & 1) + ((iota >> 2) & 1)
            + ((iota >> 3) & 1))
    ffs4 = jnp.where((iota & 1) == 1, 0,
                     jnp.where((iota & 2) == 2, 1,
                               jnp.where((iota & 4) == 4, 2, 3)))

    def row_body(r, carry):
        pvs = [rec_v[r, pl.ds(16 * b, 16)] for b in range(4)]
        qvs = [rec_v[r, pl.ds(64 + 16 * b, 16)] for b in range(2)]
        total = rec_v[r, pl.ds(HW + K_MAX, 16)][0]
        for kk in range(K_MAX // 16):
            c = iota + (16 * kk)
            wv = rec_v[r, pl.ds(HW + 16 * kk, 16)] - 1
            low = wv & 15
            rq = jnp.where(wv >> 4 == 0, _vgather(qvs[0], low),
                           _vgather(qvs[1], low))
            r0 = rq & 127
            r1 = jnp.minimum(r0 + ((rq >> 7) & 31), 64)
            r2 = jnp.minimum(r1 + ((rq >> 12) & 31), 64)
            r3 = jnp.minimum(r2 + ((rq >> 17) & 31), 64)
            k1 = r1 <= c
            k2 = r2 <= c
            k3 = r3 <= c
            e_off = (jnp.where(k1, 1, 0) + jnp.where(k2, 1, 0)
                     + jnp.where(k3, 1, 0))
            r_sel = jnp.where(k3, r3, jnp.where(k2, r2, jnp.where(k1, r1, r0)))
            whw = (wv << 2) + e_off
            c_loc = c - r_sel
            u = whw >> 1
            ulow = u & 15
            uhi = u >> 4
            gp = jnp.zeros((16,), jnp.int32)
            for b in range(4):
                gp = jnp.where(uhi == b, _vgather(pvs[b], ulow), gp)
            vv = jnp.where((whw & 1) == 1, (gp >> 16) & 0xFFFF, gp & 0xFFFF)
            n0 = vv & 15
            n1 = (vv >> 4) & 15
            n2 = (vv >> 8) & 15
            n3 = (vv >> 12) & 15
            s1 = _vgather(pop4, n0)
            s2 = s1 + _vgather(pop4, n1)
            s3 = s2 + _vgather(pop4, n2)
            k1b = c_loc >= s1
            k2b = c_loc >= s2
            k3b = c_loc >= s3
            nib = jnp.where(k3b, n3, jnp.where(k2b, n2, jnp.where(k1b, n1, n0)))
            nbase = (jnp.where(k1b, 1, 0) + jnp.where(k2b, 1, 0)
                     + jnp.where(k3b, 1, 0)) << 2
            rem = c_loc - jnp.where(k3b, s3, jnp.where(k2b, s2,
                                                       jnp.where(k1b, s1, 0)))
            x = nib
            x = jnp.where(rem >= 1, x & (x - 1), x)
            x = jnp.where(rem >= 2, x & (x - 1), x)
            x = jnp.where(rem >= 3, x & (x - 1), x)
            pos = nbase + _vgather(ffs4, x)
            res = jnp.where(c < total, whw * 16 + pos, -1)
            out_v[r, pl.ds(16 * kk, 16)] = res
        return carry

    lax.fori_loop(0, RPW, row_body, jnp.int32(0))
    pltpu.sync_copy(out_v, out_hbm.at[pl.ds(base, RPW)])


def kernel(query_up, key_up, head_idx):
    del head_idx
    sel = _build_select()
    outs = []
    for b in range(B):
        rec = _match_records(query_up[b], key_up[b])
        outs.append(sel(rec))
    return jnp.stack(outs)
